# baseline (speedup 1.0000x reference)
import jax
import jax.numpy as jnp
from jax.experimental import pallas as pl
from jax.experimental.pallas import tpu as pltpu

_NEG_SLOPE = 0.1
_CHUNK = 2000
_NBUF = 6


def _prelu(x):
    return jnp.where(x >= 0, x, _NEG_SLOPE * x)


def _body(src_hbm, snk_hbm, osrc_hbm, osnk_hbm,
          xb, yb, oxb, oyb, sems):
    n = src_hbm.shape[0]
    nchunk = n // _CHUNK

    def start_in(c, s):
        sl = pl.ds(c * _CHUNK, _CHUNK)
        pltpu.make_async_copy(src_hbm.at[sl, :], xb.at[s], sems.at[0, s]).start()
        pltpu.make_async_copy(snk_hbm.at[sl, :], yb.at[s], sems.at[1, s]).start()

    def wait_in(s):
        pltpu.make_async_copy(src_hbm.at[pl.ds(0, _CHUNK), :], xb.at[s], sems.at[0, s]).wait()
        pltpu.make_async_copy(snk_hbm.at[pl.ds(0, _CHUNK), :], yb.at[s], sems.at[1, s]).wait()

    def start_out(c, s):
        sl = pl.ds(c * _CHUNK, _CHUNK)
        pltpu.make_async_copy(oxb.at[s], osrc_hbm.at[sl, :], sems.at[2, s]).start()
        pltpu.make_async_copy(oyb.at[s], osnk_hbm.at[sl, :], sems.at[3, s]).start()

    def wait_out(s):
        pltpu.make_async_copy(oxb.at[s], osrc_hbm.at[pl.ds(0, _CHUNK), :], sems.at[2, s]).wait()
        pltpu.make_async_copy(oyb.at[s], osnk_hbm.at[pl.ds(0, _CHUNK), :], sems.at[3, s]).wait()

    for c in range(_NBUF):
        start_in(c, c)
    for c in range(nchunk):
        s = c % _NBUF
        wait_in(s)
        if c >= _NBUF:
            wait_out(s)
        oxb[s] = _prelu(xb[s])
        oyb[s] = _prelu(yb[s])
        start_out(c, s)
        if c + _NBUF < nchunk:
            start_in(c + _NBUF, s)
    for c in range(nchunk - _NBUF, nchunk):
        wait_out(c % _NBUF)


def kernel(source_emb, sink_emb, source_edge_index, sink_edge_index,
           W_src, a_src_s, a_src_d, b_src,
           W_snk, a_snk_s, a_snk_d, b_snk):
    n, d = source_emb.shape
    hbm = pl.BlockSpec(memory_space=pltpu.MemorySpace.HBM)
    out_src, out_snk = pl.pallas_call(
        _body,
        in_specs=[hbm, hbm],
        out_specs=[hbm, hbm],
        out_shape=[
            jax.ShapeDtypeStruct((n, d), source_emb.dtype),
            jax.ShapeDtypeStruct((n, d), sink_emb.dtype),
        ],
        scratch_shapes=[
            pltpu.VMEM((_NBUF, _CHUNK, d), jnp.float32),
            pltpu.VMEM((_NBUF, _CHUNK, d), jnp.float32),
            pltpu.VMEM((_NBUF, _CHUNK, d), jnp.float32),
            pltpu.VMEM((_NBUF, _CHUNK, d), jnp.float32),
            pltpu.SemaphoreType.DMA((4, _NBUF)),
        ],
    )(source_emb, sink_emb)
    return (out_src, out_snk)


# speedup vs baseline: 9.3005x; 9.3005x over previous
import jax
import jax.numpy as jnp
from jax.experimental import pallas as pl

_NEG_SLOPE = 0.1
_BLOCK_COLS = 8192


def _prelu_body(src_ref, snk_ref, out_src_ref, out_snk_ref):
    x = src_ref[...]
    out_src_ref[...] = jnp.where(x >= 0, x, _NEG_SLOPE * x)
    y = snk_ref[...]
    out_snk_ref[...] = jnp.where(y >= 0, y, _NEG_SLOPE * y)


def kernel(source_emb, sink_emb, source_edge_index, sink_edge_index,
           W_src, a_src_s, a_src_d, b_src,
           W_snk, a_snk_s, a_snk_d, b_snk):
    n, d = source_emb.shape
    src = source_emb.T
    snk = sink_emb.T
    grid = (pl.cdiv(n, _BLOCK_COLS),)
    spec = pl.BlockSpec((d, _BLOCK_COLS), lambda i: (0, i))
    out_src, out_snk = pl.pallas_call(
        _prelu_body,
        grid=grid,
        in_specs=[spec, spec],
        out_specs=[spec, spec],
        out_shape=[
            jax.ShapeDtypeStruct((d, n), source_emb.dtype),
            jax.ShapeDtypeStruct((d, n), sink_emb.dtype),
        ],
    )(src, snk)
    return (out_src.T, out_snk.T)
